# baseline (speedup 1.0000x reference)
import dataclasses
import functools

import jax
import jax.numpy as jnp
from jax import lax
from jax.experimental import pallas as pl
from jax.experimental.pallas import tpu as pltpu
from jax.experimental.pallas import tpu_sc as plsc

N = 10000
NP = 10240
E = 320000
D = 128

NC = 2
NS = 16
NW = NC * NS
EW = E // NW
G = 80
NCH = EW // G
NGRP = 5
GRPC = NCH // NGRP
RPS = NP // NS

_vmesh = plsc.VectorSubcoreMesh(core_axis_name="c", subcore_axis_name="s",
                                num_cores=NC, num_subcores=NS)

_sc_params = pltpu.CompilerParams()
if "needs_layout_passes" in pltpu.CompilerParams.__dataclass_fields__:
    _sc_params = dataclasses.replace(_sc_params, needs_layout_passes=False)


@jax.jit
def _sc_degrees(e5):
    @functools.partial(
        pl.kernel,
        out_type=jax.ShapeDtypeStruct((2, NC, NS, NP), jnp.float32),
        mesh=_vmesh,
        scratch_types=[
            pltpu.VMEM((NCH, G), jnp.int32),
            pltpu.VMEM((NCH, G), jnp.int32),
            pltpu.VMEM((NP,), jnp.float32),
            pltpu.VMEM((NP,), jnp.float32),
        ],
        compiler_params=_sc_params,
    )
    def k(e_hbm, out_hbm, src_v, dst_v, hs_v, hd_v):
        cid = lax.axis_index("c")
        sid = lax.axis_index("s")

        @pl.loop(0, NP // 16)
        def _(i):
            off = pl.multiple_of(i * 16, 16)
            hs_v[pl.ds(off, 16)] = jnp.zeros((16,), jnp.float32)
            hd_v[pl.ds(off, 16)] = jnp.zeros((16,), jnp.float32)

        pltpu.sync_copy(e_hbm.at[0, cid, sid], src_v)
        pltpu.sync_copy(e_hbm.at[1, cid, sid], dst_v)

        @pl.loop(0, NCH)
        def _(j):
            for kk in range(G // 16):
                s_idx = src_v[j, pl.ds(kk * 16, 16)]
                d_idx = dst_v[j, pl.ds(kk * 16, 16)]
                plsc.addupdate_scatter(hs_v, [s_idx], jnp.ones((16,), jnp.float32))
                plsc.addupdate_scatter(hd_v, [d_idx], jnp.ones((16,), jnp.float32))

        pltpu.sync_copy(hs_v, out_hbm.at[0, cid, sid])
        pltpu.sync_copy(hd_v, out_hbm.at[1, cid, sid])

    return k(e5)


@jax.jit
def _sc_aggregate(y, e6):
    @functools.partial(
        pl.kernel,
        out_type=jax.ShapeDtypeStruct((NC, NP, D), jnp.float32),
        mesh=_vmesh,
        scratch_types=[
            pltpu.VMEM((2, GRPC, G), jnp.int32),
            pltpu.VMEM((2, GRPC, G), jnp.int32),
            pltpu.VMEM((3, G, D), jnp.float32),
            pltpu.VMEM_SHARED((NP, D), jnp.float32),
            pltpu.SemaphoreType.DMA,
            pltpu.SemaphoreType.DMA,
            pltpu.SemaphoreType.DMA,
        ],
        compiler_params=_sc_params,
    )
    def k(y_hbm, e_hbm, out_hbm, src_v, dst_v, rows_v, acc_sh, gsem, ssem,
          isem):
        cid = lax.axis_index("c")
        sid = lax.axis_index("s")
        base = sid * RPS

        @pl.loop(0, G)
        def _(i):
            for kk in range(D // 16):
                rows_v[0, i, pl.ds(kk * 16, 16)] = jnp.zeros((16,), jnp.float32)

        for kk in range(RPS // G):
            pltpu.sync_copy(rows_v.at[0], acc_sh.at[pl.ds(base + kk * G, G)])
        plsc.subcore_barrier()

        pltpu.sync_copy(e_hbm.at[0, cid, sid, 0], src_v.at[0])
        pltpu.sync_copy(e_hbm.at[1, cid, sid, 0], dst_v.at[0])

        pltpu.async_copy(y_hbm.at[src_v.at[0, 0]], rows_v.at[0], gsem)
        pltpu.async_copy(y_hbm.at[src_v.at[0, 1]], rows_v.at[1], gsem)

        @pl.loop(0, NGRP)
        def _(g):
            p = lax.rem(g, 2)
            pn = 1 - p
            q0 = g * GRPC

            @pl.when(g + 1 < NGRP)
            def _():
                pltpu.async_copy(e_hbm.at[0, cid, sid, g + 1], src_v.at[pn],
                                 isem)
                pltpu.async_copy(e_hbm.at[1, cid, sid, g + 1], dst_v.at[pn],
                                 isem)

            @pl.loop(0, GRPC - 2)
            def _(j):
                b = lax.rem(q0 + j, 3)
                pltpu.make_async_copy(y_hbm.at[src_v.at[p, j]],
                                      rows_v.at[b], gsem).wait()
                pltpu.async_copy(y_hbm.at[src_v.at[p, j + 2]],
                                 rows_v.at[lax.rem(q0 + j + 2, 3)], gsem)
                pltpu.async_copy(rows_v.at[b], acc_sh.at[dst_v.at[p, j]],
                                 ssem, add=True).wait()

            @pl.when(g + 1 < NGRP)
            def _():
                pltpu.make_async_copy(e_hbm.at[0, cid, sid, 0], src_v.at[pn],
                                      isem).wait()
                pltpu.make_async_copy(e_hbm.at[1, cid, sid, 0], dst_v.at[pn],
                                      isem).wait()

                @pl.loop(GRPC - 2, GRPC)
                def _(j):
                    b = lax.rem(q0 + j, 3)
                    pltpu.make_async_copy(y_hbm.at[src_v.at[p, j]],
                                          rows_v.at[b], gsem).wait()
                    pltpu.async_copy(y_hbm.at[src_v.at[pn, j - (GRPC - 2)]],
                                     rows_v.at[lax.rem(q0 + j + 2, 3)], gsem)
                    pltpu.async_copy(rows_v.at[b], acc_sh.at[dst_v.at[p, j]],
                                     ssem, add=True).wait()

            @pl.when(g + 1 >= NGRP)
            def _():
                @pl.loop(GRPC - 2, GRPC)
                def _(j):
                    b = lax.rem(q0 + j, 3)
                    pltpu.make_async_copy(y_hbm.at[src_v.at[p, j]],
                                          rows_v.at[b], gsem).wait()
                    pltpu.async_copy(rows_v.at[b], acc_sh.at[dst_v.at[p, j]],
                                     ssem, add=True).wait()

        plsc.subcore_barrier()
        pltpu.sync_copy(acc_sh.at[pl.ds(base, RPS)],
                        out_hbm.at[cid, pl.ds(base, RPS)])

    return k(y, e6)


_BM = 1024


@jax.jit
def _tc_matmul_raw(x, w):
    def body(x_ref, w_ref, y_ref):
        y_ref[...] = jnp.dot(x_ref[...], w_ref[...],
                             preferred_element_type=jnp.float32)

    return pl.pallas_call(
        body,
        grid=(NP // _BM,),
        in_specs=[
            pl.BlockSpec((_BM, D), lambda i: (i, 0)),
            pl.BlockSpec((D, D), lambda i: (0, 0)),
        ],
        out_specs=pl.BlockSpec((_BM, D), lambda i: (i, 0)),
        out_shape=jax.ShapeDtypeStruct((NP, D), jnp.float32),
    )(x, w)


@jax.jit
def _tc_prep(histp, yraw):
    def body(dop_ref, dip_ref, yr_ref, y_ref, no_ref, ni_ref):
        do = jnp.sum(dop_ref[0], axis=0, keepdims=True)
        di = jnp.sum(dip_ref[0], axis=0, keepdims=True)
        no = jnp.transpose(jnp.where(do > 0, lax.rsqrt(do), 0.0))
        ni = jnp.transpose(jnp.where(di > 0, lax.rsqrt(di), 0.0))
        no_ref[...] = no
        ni_ref[...] = ni
        y_ref[...] = yr_ref[...] * no

    return pl.pallas_call(
        body,
        grid=(NP // _BM,),
        in_specs=[
            pl.BlockSpec((1, NW, _BM), lambda i: (0, 0, i)),
            pl.BlockSpec((1, NW, _BM), lambda i: (1, 0, i)),
            pl.BlockSpec((_BM, D), lambda i: (i, 0)),
        ],
        out_specs=(
            pl.BlockSpec((_BM, D), lambda i: (i, 0)),
            pl.BlockSpec((_BM, 1), lambda i: (i, 0)),
            pl.BlockSpec((_BM, 1), lambda i: (i, 0)),
        ),
        out_shape=(
            jax.ShapeDtypeStruct((NP, D), jnp.float32),
            jax.ShapeDtypeStruct((NP, 1), jnp.float32),
            jax.ShapeDtypeStruct((NP, 1), jnp.float32),
        ),
    )(histp, histp, yraw)


@jax.jit
def _tc_mid(acc, n_in, b, w, n_out):
    def body(a0_ref, a1_ref, ni_ref, b_ref, w_ref, no_ref, y_ref):
        h = (a0_ref[0] + a1_ref[0]) * ni_ref[...] + b_ref[...]
        h = jnp.maximum(h, 0.0)
        y_ref[...] = jnp.dot(h, w_ref[...],
                             preferred_element_type=jnp.float32) * no_ref[...]

    return pl.pallas_call(
        body,
        grid=(NP // _BM,),
        in_specs=[
            pl.BlockSpec((1, _BM, D), lambda i: (0, i, 0)),
            pl.BlockSpec((1, _BM, D), lambda i: (1, i, 0)),
            pl.BlockSpec((_BM, 1), lambda i: (i, 0)),
            pl.BlockSpec((1, D), lambda i: (0, 0)),
            pl.BlockSpec((D, D), lambda i: (0, 0)),
            pl.BlockSpec((_BM, 1), lambda i: (i, 0)),
        ],
        out_specs=pl.BlockSpec((_BM, D), lambda i: (i, 0)),
        out_shape=jax.ShapeDtypeStruct((NP, D), jnp.float32),
    )(acc, acc, n_in, b, w, n_out)


_BMF = 1000


@jax.jit
def _tc_final(acc, n_in, b):
    def body(a0_ref, a1_ref, ni_ref, b_ref, y_ref):
        y_ref[...] = (a0_ref[0] + a1_ref[0]) * ni_ref[...] + b_ref[...]

    return pl.pallas_call(
        body,
        grid=(N // _BMF,),
        in_specs=[
            pl.BlockSpec((1, _BMF, D), lambda i: (0, i, 0)),
            pl.BlockSpec((1, _BMF, D), lambda i: (1, i, 0)),
            pl.BlockSpec((_BMF, 1), lambda i: (i, 0)),
            pl.BlockSpec((1, D), lambda i: (0, 0)),
        ],
        out_specs=pl.BlockSpec((_BMF, D), lambda i: (i, 0)),
        out_shape=jax.ShapeDtypeStruct((N, D), jnp.float32),
    )(acc, acc, n_in, b)


@jax.jit
def kernel(x, edge_index, W1, b1, W2, b2):
    e5 = edge_index.reshape(2, NC, NS, NCH, G)
    e6 = edge_index.reshape(2, NC, NS, NGRP, GRPC, G)

    hist = _sc_degrees(e5)
    xp = jnp.pad(x, ((0, NP - N), (0, 0)))

    yraw = _tc_matmul_raw(xp, W1)
    y1, n_out, n_in = _tc_prep(hist.reshape(2, NW, NP), yraw)
    acc1 = _sc_aggregate(y1, e6)
    y2 = _tc_mid(acc1, n_in, b1.reshape(1, D), W2, n_out)
    acc2 = _sc_aggregate(y2, e6)
    return _tc_final(acc2, n_in, b2.reshape(1, D))


# speedup vs baseline: 1.0261x; 1.0261x over previous
import dataclasses
import functools

import jax
import jax.numpy as jnp
from jax import lax
from jax.experimental import pallas as pl
from jax.experimental.pallas import tpu as pltpu
from jax.experimental.pallas import tpu_sc as plsc

N = 10000
NP = 10240
E = 320000
D = 128

NC = 2
NS = 16
NW = NC * NS
EW = E // NW
G = 80
NCH = EW // G
NGRP = 5
GRPC = NCH // NGRP
RPS = NP // NS

_vmesh = plsc.VectorSubcoreMesh(core_axis_name="c", subcore_axis_name="s",
                                num_cores=NC, num_subcores=NS)

_sc_params = pltpu.CompilerParams()
if "needs_layout_passes" in pltpu.CompilerParams.__dataclass_fields__:
    _sc_params = dataclasses.replace(_sc_params, needs_layout_passes=False)


@jax.jit
def _sc_degrees(e5):
    @functools.partial(
        pl.kernel,
        out_type=jax.ShapeDtypeStruct((2, NC, NS, NP), jnp.float32),
        mesh=_vmesh,
        scratch_types=[
            pltpu.VMEM((NCH, G), jnp.int32),
            pltpu.VMEM((NCH, G), jnp.int32),
            pltpu.VMEM((NP,), jnp.float32),
            pltpu.VMEM((NP,), jnp.float32),
        ],
        compiler_params=_sc_params,
    )
    def k(e_hbm, out_hbm, src_v, dst_v, hs_v, hd_v):
        cid = lax.axis_index("c")
        sid = lax.axis_index("s")

        @pl.loop(0, NP // 16)
        def _(i):
            off = pl.multiple_of(i * 16, 16)
            hs_v[pl.ds(off, 16)] = jnp.zeros((16,), jnp.float32)
            hd_v[pl.ds(off, 16)] = jnp.zeros((16,), jnp.float32)

        pltpu.sync_copy(e_hbm.at[0, cid, sid], src_v)
        pltpu.sync_copy(e_hbm.at[1, cid, sid], dst_v)

        @pl.loop(0, NCH)
        def _(j):
            for kk in range(G // 16):
                s_idx = src_v[j, pl.ds(kk * 16, 16)]
                d_idx = dst_v[j, pl.ds(kk * 16, 16)]
                plsc.addupdate_scatter(hs_v, [s_idx], jnp.ones((16,), jnp.float32))
                plsc.addupdate_scatter(hd_v, [d_idx], jnp.ones((16,), jnp.float32))

        pltpu.sync_copy(hs_v, out_hbm.at[0, cid, sid])
        pltpu.sync_copy(hd_v, out_hbm.at[1, cid, sid])

    return k(e5)


@jax.jit
def _sc_aggregate(y, e6):
    @functools.partial(
        pl.kernel,
        out_type=jax.ShapeDtypeStruct((NC, NP, D), jnp.float32),
        mesh=_vmesh,
        scratch_types=[
            pltpu.VMEM((2, GRPC, G), jnp.int32),
            pltpu.VMEM((2, GRPC, G), jnp.int32),
            pltpu.VMEM((3, G, D), jnp.float32),
            pltpu.VMEM_SHARED((NP, D), jnp.float32),
            pltpu.SemaphoreType.DMA,
            pltpu.SemaphoreType.DMA,
            pltpu.SemaphoreType.DMA,
        ],
        compiler_params=_sc_params,
    )
    def k(y_hbm, e_hbm, out_hbm, src_v, dst_v, rows_v, acc_sh, gsem, ssem,
          isem):
        cid = lax.axis_index("c")
        sid = lax.axis_index("s")
        base = sid * RPS

        @pl.loop(0, G)
        def _(i):
            for kk in range(D // 16):
                rows_v[0, i, pl.ds(kk * 16, 16)] = jnp.zeros((16,), jnp.float32)

        for kk in range(RPS // G):
            pltpu.sync_copy(rows_v.at[0], acc_sh.at[pl.ds(base + kk * G, G)])
        plsc.subcore_barrier()

        pltpu.sync_copy(e_hbm.at[0, cid, sid, 0], src_v.at[0])
        pltpu.sync_copy(e_hbm.at[1, cid, sid, 0], dst_v.at[0])

        pltpu.async_copy(y_hbm.at[src_v.at[0, 0]], rows_v.at[0], gsem)
        pltpu.async_copy(y_hbm.at[src_v.at[0, 1]], rows_v.at[1], gsem)

        @pl.loop(0, NGRP)
        def _(g):
            p = lax.rem(g, 2)
            pn = 1 - p
            q0 = g * GRPC

            @pl.when(g + 1 < NGRP)
            def _():
                pltpu.async_copy(e_hbm.at[0, cid, sid, g + 1], src_v.at[pn],
                                 isem)
                pltpu.async_copy(e_hbm.at[1, cid, sid, g + 1], dst_v.at[pn],
                                 isem)

            @pl.loop(0, GRPC - 2)
            def _(j):
                b = lax.rem(q0 + j, 3)
                pltpu.make_async_copy(y_hbm.at[src_v.at[p, j]],
                                      rows_v.at[b], gsem).wait()
                pltpu.async_copy(y_hbm.at[src_v.at[p, j + 2]],
                                 rows_v.at[lax.rem(q0 + j + 2, 3)], gsem)
                pltpu.async_copy(rows_v.at[b], acc_sh.at[dst_v.at[p, j]],
                                 ssem, add=True).wait()

            @pl.when(g + 1 < NGRP)
            def _():
                pltpu.make_async_copy(e_hbm.at[0, cid, sid, 0], src_v.at[pn],
                                      isem).wait()
                pltpu.make_async_copy(e_hbm.at[1, cid, sid, 0], dst_v.at[pn],
                                      isem).wait()

                @pl.loop(GRPC - 2, GRPC)
                def _(j):
                    b = lax.rem(q0 + j, 3)
                    pltpu.make_async_copy(y_hbm.at[src_v.at[p, j]],
                                          rows_v.at[b], gsem).wait()
                    pltpu.async_copy(y_hbm.at[src_v.at[pn, j - (GRPC - 2)]],
                                     rows_v.at[lax.rem(q0 + j + 2, 3)], gsem)
                    pltpu.async_copy(rows_v.at[b], acc_sh.at[dst_v.at[p, j]],
                                     ssem, add=True).wait()

            @pl.when(g + 1 >= NGRP)
            def _():
                @pl.loop(GRPC - 2, GRPC)
                def _(j):
                    b = lax.rem(q0 + j, 3)
                    pltpu.make_async_copy(y_hbm.at[src_v.at[p, j]],
                                          rows_v.at[b], gsem).wait()
                    pltpu.async_copy(rows_v.at[b], acc_sh.at[dst_v.at[p, j]],
                                     ssem, add=True).wait()

        plsc.subcore_barrier()
        pltpu.sync_copy(acc_sh.at[pl.ds(base, RPS)],
                        out_hbm.at[cid, pl.ds(base, RPS)])

    return k(y, e6)


_BM = 1024


@jax.jit
def _tc_prep(histp, x, w):
    def body(dop_ref, dip_ref, x_ref, w_ref, y_ref, no_ref, ni_ref):
        do = jnp.sum(dop_ref[0], axis=0, keepdims=True)
        di = jnp.sum(dip_ref[0], axis=0, keepdims=True)
        no = jnp.transpose(jnp.where(do > 0, lax.rsqrt(do), 0.0))
        ni = jnp.transpose(jnp.where(di > 0, lax.rsqrt(di), 0.0))
        no_ref[...] = no
        ni_ref[...] = ni
        y_ref[...] = jnp.dot(x_ref[...], w_ref[...],
                             preferred_element_type=jnp.float32) * no

    return pl.pallas_call(
        body,
        grid=(NP // _BM,),
        in_specs=[
            pl.BlockSpec((1, NW, _BM), lambda i: (0, 0, i)),
            pl.BlockSpec((1, NW, _BM), lambda i: (1, 0, i)),
            pl.BlockSpec((_BM, D), lambda i: (i, 0)),
            pl.BlockSpec((D, D), lambda i: (0, 0)),
        ],
        out_specs=(
            pl.BlockSpec((_BM, D), lambda i: (i, 0)),
            pl.BlockSpec((_BM, 1), lambda i: (i, 0)),
            pl.BlockSpec((_BM, 1), lambda i: (i, 0)),
        ),
        out_shape=(
            jax.ShapeDtypeStruct((NP, D), jnp.float32),
            jax.ShapeDtypeStruct((NP, 1), jnp.float32),
            jax.ShapeDtypeStruct((NP, 1), jnp.float32),
        ),
    )(histp, histp, x, w)


@jax.jit
def _tc_mid(acc, n_in, b, w, n_out):
    def body(a0_ref, a1_ref, ni_ref, b_ref, w_ref, no_ref, y_ref):
        h = (a0_ref[0] + a1_ref[0]) * ni_ref[...] + b_ref[...]
        h = jnp.maximum(h, 0.0)
        y_ref[...] = jnp.dot(h, w_ref[...],
                             preferred_element_type=jnp.float32) * no_ref[...]

    return pl.pallas_call(
        body,
        grid=(NP // _BM,),
        in_specs=[
            pl.BlockSpec((1, _BM, D), lambda i: (0, i, 0)),
            pl.BlockSpec((1, _BM, D), lambda i: (1, i, 0)),
            pl.BlockSpec((_BM, 1), lambda i: (i, 0)),
            pl.BlockSpec((1, D), lambda i: (0, 0)),
            pl.BlockSpec((D, D), lambda i: (0, 0)),
            pl.BlockSpec((_BM, 1), lambda i: (i, 0)),
        ],
        out_specs=pl.BlockSpec((_BM, D), lambda i: (i, 0)),
        out_shape=jax.ShapeDtypeStruct((NP, D), jnp.float32),
    )(acc, acc, n_in, b, w, n_out)


_BMF = 1000


@jax.jit
def _tc_final(acc, n_in, b):
    def body(a0_ref, a1_ref, ni_ref, b_ref, y_ref):
        y_ref[...] = (a0_ref[0] + a1_ref[0]) * ni_ref[...] + b_ref[...]

    return pl.pallas_call(
        body,
        grid=(N // _BMF,),
        in_specs=[
            pl.BlockSpec((1, _BMF, D), lambda i: (0, i, 0)),
            pl.BlockSpec((1, _BMF, D), lambda i: (1, i, 0)),
            pl.BlockSpec((_BMF, 1), lambda i: (i, 0)),
            pl.BlockSpec((1, D), lambda i: (0, 0)),
        ],
        out_specs=pl.BlockSpec((_BMF, D), lambda i: (i, 0)),
        out_shape=jax.ShapeDtypeStruct((N, D), jnp.float32),
    )(acc, acc, n_in, b)


@jax.jit
def kernel(x, edge_index, W1, b1, W2, b2):
    e5 = edge_index.reshape(2, NC, NS, NCH, G)
    e6 = edge_index.reshape(2, NC, NS, NGRP, GRPC, G)

    hist = _sc_degrees(e5)
    xp = jnp.pad(x, ((0, NP - N), (0, 0)))

    y1, n_out, n_in = _tc_prep(hist.reshape(2, NW, NP), xp, W1)
    acc1 = _sc_aggregate(y1, e6)
    y2 = _tc_mid(acc1, n_in, b1.reshape(1, D), W2, n_out)
    acc2 = _sc_aggregate(y2, e6)
    return _tc_final(acc2, n_in, b2.reshape(1, D))
